# Initial kernel scaffold; baseline (speedup 1.0000x reference)
import jax
import jax.numpy as jnp
from jax.experimental import pallas as pl


def kernel(x, pos_table):
    raise NotImplementedError("write your pallas kernel here")


# speedup vs baseline: 2.2657x; 2.2657x over previous
import jax
import jax.numpy as jnp
from jax.experimental import pallas as pl
from jax.experimental.pallas import tpu as pltpu

_BLOCK_S = 512


def _add_kernel(x_ref, pos_ref, out_ref):
    out_ref[0] = x_ref[0] + pos_ref[...]


def kernel(x, pos_table):
    batch, seq_len, d_model = x.shape
    bs = _BLOCK_S
    grid = (seq_len // bs, batch)
    return pl.pallas_call(
        _add_kernel,
        grid=grid,
        in_specs=[
            pl.BlockSpec((1, bs, d_model), lambda i, j: (j, i, 0)),
            pl.BlockSpec((bs, d_model), lambda i, j: (i, 0)),
        ],
        out_specs=pl.BlockSpec((1, bs, d_model), lambda i, j: (j, i, 0)),
        out_shape=jax.ShapeDtypeStruct(x.shape, x.dtype),
        compiler_params=pltpu.CompilerParams(
            dimension_semantics=("parallel", "parallel"),
        ),
    )(x, pos_table)


# speedup vs baseline: 2.5224x; 1.1133x over previous
import jax
import jax.numpy as jnp
from jax.experimental import pallas as pl
from jax.experimental.pallas import tpu as pltpu

_BLOCK_S = 1024


def _add_kernel(x_ref, pos_ref, out_ref):
    out_ref[0] = x_ref[0] + pos_ref[...]


def kernel(x, pos_table):
    batch, seq_len, d_model = x.shape
    bs = _BLOCK_S
    grid = (seq_len // bs, batch)
    return pl.pallas_call(
        _add_kernel,
        grid=grid,
        in_specs=[
            pl.BlockSpec((1, bs, d_model), lambda i, j: (j, i, 0)),
            pl.BlockSpec((bs, d_model), lambda i, j: (i, 0)),
        ],
        out_specs=pl.BlockSpec((1, bs, d_model), lambda i, j: (j, i, 0)),
        out_shape=jax.ShapeDtypeStruct(x.shape, x.dtype),
        compiler_params=pltpu.CompilerParams(
            dimension_semantics=("parallel", "parallel"),
        ),
    )(x, pos_table)


# speedup vs baseline: 2.6197x; 1.0386x over previous
import jax
import jax.numpy as jnp
from jax.experimental import pallas as pl
from jax.experimental.pallas import tpu as pltpu

_BLOCK_S = 2048


def _add_kernel(x_ref, pos_ref, out_ref):
    out_ref[0] = x_ref[0] + pos_ref[...]


def kernel(x, pos_table):
    batch, seq_len, d_model = x.shape
    bs = _BLOCK_S
    grid = (seq_len // bs, batch)
    return pl.pallas_call(
        _add_kernel,
        grid=grid,
        in_specs=[
            pl.BlockSpec((1, bs, d_model), lambda i, j: (j, i, 0)),
            pl.BlockSpec((bs, d_model), lambda i, j: (i, 0)),
        ],
        out_specs=pl.BlockSpec((1, bs, d_model), lambda i, j: (j, i, 0)),
        out_shape=jax.ShapeDtypeStruct(x.shape, x.dtype),
        compiler_params=pltpu.CompilerParams(
            dimension_semantics=("parallel", "parallel"),
        ),
    )(x, pos_table)


# speedup vs baseline: 2.6253x; 1.0021x over previous
import jax
import jax.numpy as jnp
from jax.experimental import pallas as pl
from jax.experimental.pallas import tpu as pltpu

_BLOCK_S = 512


def _add_kernel(x_ref, pos_ref, out_ref):
    out_ref[...] = x_ref[...] + pos_ref[...][None]


def kernel(x, pos_table):
    batch, seq_len, d_model = x.shape
    bs = _BLOCK_S
    grid = (seq_len // bs,)
    return pl.pallas_call(
        _add_kernel,
        grid=grid,
        in_specs=[
            pl.BlockSpec((batch, bs, d_model), lambda i: (0, i, 0)),
            pl.BlockSpec((bs, d_model), lambda i: (i, 0)),
        ],
        out_specs=pl.BlockSpec((batch, bs, d_model), lambda i: (0, i, 0)),
        out_shape=jax.ShapeDtypeStruct(x.shape, x.dtype),
        compiler_params=pltpu.CompilerParams(
            dimension_semantics=("parallel",),
        ),
    )(x, pos_table)
